# baseline (speedup 1.0000x reference)
import functools

import jax
import jax.numpy as jnp
import numpy as np
from jax import lax
from jax.experimental import pallas as pl
from jax.experimental.pallas import tpu as pltpu
from jax.experimental.pallas import tpu_sc as plsc

_B, _S, _H = 4, 8192, 1024
_SB = 1024
_NB2 = _S // (2 * _SB)
_K = 5
_NJ = _B
_NEG = np.float32(-3.0e38)
_IMAX = np.int32(2**31 - 1)


def _proj_body(x1_ref, x2_ref, w_ref, mask_ref, out_ref, lse_ref, m_scr, s_scr):
    i = pl.program_id(1)
    w = w_ref[...]
    la = lax.dot_general(w, x1_ref[0], (((1,), (1,)), ((), ())),
                         preferred_element_type=jnp.float32)
    lb = lax.dot_general(w, x2_ref[0], (((1,), (1,)), ((), ())),
                         preferred_element_type=jnp.float32)
    logits = jnp.concatenate([la, lb], axis=1)
    mask = mask_ref[0]
    logits = logits + (1.0 - mask) * np.float32(-1e30)
    out_ref[0] = logits

    bm = jnp.max(logits, axis=1, keepdims=True)
    bs = jnp.sum(jnp.exp(logits - bm), axis=1, keepdims=True)
    bm = jnp.broadcast_to(bm, (2, 128))
    bs = jnp.broadcast_to(bs, (2, 128))

    @pl.when(i == 0)
    def _():
        m_scr[0:2, :] = bm
        s_scr[0:2, :] = bs

    @pl.when(i > 0)
    def _():
        m_old = m_scr[0:2, :]
        s_old = s_scr[0:2, :]
        m_new = jnp.maximum(m_old, bm)
        s_new = s_old * jnp.exp(m_old - m_new) + bs * jnp.exp(bm - m_new)
        m_scr[0:2, :] = m_new
        s_scr[0:2, :] = s_new

    @pl.when(i == _NB2 - 1)
    def _():
        lse_ref[0] = m_scr[0:2, :] + jnp.log(s_scr[0:2, :])


def _project(seq_output, w2, mask3, off):
    return pl.pallas_call(
        _proj_body,
        grid=(_B // 2, _NB2),
        in_specs=[
            pl.BlockSpec((1, _SB, _H), lambda b, i, off=off: (b + off, 2 * i, 0)),
            pl.BlockSpec((1, _SB, _H),
                         lambda b, i, off=off: (b + off, 2 * i + 1, 0)),
            pl.BlockSpec((2, _H), lambda b, i: (0, 0)),
            pl.BlockSpec((1, 1, 2 * _SB), lambda b, i, off=off: (b + off, 0, i)),
        ],
        out_specs=[
            pl.BlockSpec((1, 2, 2 * _SB), lambda b, i: (b, 0, i)),
            pl.BlockSpec((1, 2, 128), lambda b, i: (b, 0, 0)),
        ],
        out_shape=[
            jax.ShapeDtypeStruct((_B // 2, 2, _S), jnp.float32),
            jax.ShapeDtypeStruct((_B // 2, 2, 128), jnp.float32),
        ],
        scratch_shapes=[
            pltpu.VMEM((8, 128), jnp.float32),
            pltpu.VMEM((8, 128), jnp.float32),
        ],
        compiler_params=pltpu.CompilerParams(
            dimension_semantics=("parallel", "arbitrary")),
    )(seq_output, seq_output, w2, mask3)


def _topk_sc_body(logits_hbm, outv_hbm, outi_hbm, row_v, vout_v, iout_v):
    cid = lax.axis_index("c")
    sid = lax.axis_index("s")
    wid = sid * 2 + cid

    @pl.when(wid < _NJ)
    def _():
        pltpu.sync_copy(logits_hbm.at[wid], row_v)
        lane = lax.iota(jnp.int32, 16)
        U = 8

        def body(j, carry):
            ts = list(carry[0])
            js = list(carry[1])
            for u in range(U):
                base = (j * U + u) * 16
                v = row_v[pl.ds(base, 16)]
                gi = base + lane
                for r in range(_K):
                    upd = v > ts[r]
                    tv, ti = ts[r], js[r]
                    ts[r] = jnp.where(upd, v, tv)
                    js[r] = jnp.where(upd, gi, ti)
                    v = jnp.where(upd, tv, v)
                    gi = jnp.where(upd, ti, gi)
            return tuple(ts), tuple(js)

        init = (tuple(jnp.full((16,), _NEG, jnp.float32) for _ in range(_K)),
                tuple(jnp.zeros((16,), jnp.int32) for _ in range(_K)))
        ts, js = lax.fori_loop(0, _S // 16 // U, body, init)
        ts, js = list(ts), list(js)

        topv = jnp.full((16,), _NEG, jnp.float32)
        topi = jnp.zeros((16,), jnp.int32)
        for r in range(_K):
            m = ts[0]
            for q in range(1, _K):
                m = jnp.maximum(m, ts[q])
            smax = jnp.max(m)
            cand = jnp.full((16,), _IMAX, jnp.int32)
            for q in range(_K):
                cand = jnp.minimum(cand, jnp.where(ts[q] == smax, js[q], _IMAX))
            widx = jnp.min(cand)
            for q in range(_K):
                ts[q] = jnp.where(js[q] == widx, _NEG, ts[q])
            topv = jnp.where(lane == r, smax, topv)
            topi = jnp.where(lane == r, widx, topi)
        vout_v[...] = topv
        iout_v[...] = topi
        pltpu.sync_copy(vout_v, outv_hbm.at[wid])
        pltpu.sync_copy(iout_v, outi_hbm.at[wid])


@functools.cache
def _topk_sc():
    return functools.partial(
        pl.kernel,
        mesh=plsc.VectorSubcoreMesh(core_axis_name="c", subcore_axis_name="s"),
        out_type=[
            jax.ShapeDtypeStruct((_NJ, 16), jnp.float32),
            jax.ShapeDtypeStruct((_NJ, 16), jnp.int32),
        ],
        scratch_types=[
            pltpu.VMEM((_S,), jnp.float32),
            pltpu.VMEM((16,), jnp.float32),
            pltpu.VMEM((16,), jnp.int32),
        ],
        compiler_params=pltpu.CompilerParams(needs_layout_passes=False),
    )(_topk_sc_body)


def kernel(seq_output, cls_output, passage_mask, start_position,
           start_w, start_b, end_w, end_b):
    w2 = jnp.concatenate([start_w, end_w], axis=1).T
    mask3 = passage_mask[:, None, :]
    l1, lse1 = _project(seq_output, w2, mask3, 0)
    l2, lse2 = _project(seq_output, w2, mask3, _B // 2)
    tv1, ti1 = _topk_sc()(l1.reshape(_NJ, _S))
    tv2, ti2 = _topk_sc()(l2.reshape(_NJ, _S))
    lse = jnp.concatenate([lse1, lse2])[:, :, 0]
    topv = jnp.concatenate([tv1, tv2])[:, :_K].reshape(_B, 2, _K)
    topi = jnp.concatenate([ti1, ti2])[:, :_K].reshape(_B, 2, _K)
    logp = topv - lse[:, :, None]
    res = jnp.stack([logp, topi.astype(jnp.float32)], axis=-1)
    return res[:, 0], res[:, 1]


# speedup vs baseline: 1.0657x; 1.0657x over previous
import functools

import jax
import jax.numpy as jnp
import numpy as np
from jax import lax
from jax.experimental import pallas as pl
from jax.experimental.pallas import tpu as pltpu
from jax.experimental.pallas import tpu_sc as plsc

_B, _S, _H = 4, 8192, 1024
_SB = 1024
_NB2 = _S // (2 * _SB)
_K = 5
_NJ = _B * 2
_NEG = np.float32(-3.0e38)
_IMAX = np.int32(2**31 - 1)


def _proj_body(x1_ref, x2_ref, w_ref, mask_ref, out_ref, lse_ref, m_scr, s_scr):
    i = pl.program_id(1)
    w = w_ref[...]
    la = lax.dot_general(w, x1_ref[0], (((1,), (1,)), ((), ())),
                         preferred_element_type=jnp.float32)
    lb = lax.dot_general(w, x2_ref[0], (((1,), (1,)), ((), ())),
                         preferred_element_type=jnp.float32)
    logits = jnp.concatenate([la, lb], axis=1)
    mask = mask_ref[0]
    logits = logits + (1.0 - mask) * np.float32(-1e30)
    out_ref[0] = logits

    bm = jnp.max(logits, axis=1, keepdims=True)
    bs = jnp.sum(jnp.exp(logits - bm), axis=1, keepdims=True)
    bm = jnp.broadcast_to(bm, (2, 128))
    bs = jnp.broadcast_to(bs, (2, 128))

    @pl.when(i == 0)
    def _():
        m_scr[0:2, :] = bm
        s_scr[0:2, :] = bs

    @pl.when(i > 0)
    def _():
        m_old = m_scr[0:2, :]
        s_old = s_scr[0:2, :]
        m_new = jnp.maximum(m_old, bm)
        s_new = s_old * jnp.exp(m_old - m_new) + bs * jnp.exp(bm - m_new)
        m_scr[0:2, :] = m_new
        s_scr[0:2, :] = s_new

    @pl.when(i == _NB2 - 1)
    def _():
        lse_ref[0] = m_scr[0:2, :] + jnp.log(s_scr[0:2, :])


def _project(seq_output, w2, mask3):
    return pl.pallas_call(
        _proj_body,
        grid=(_B, _NB2),
        in_specs=[
            pl.BlockSpec((1, _SB, _H), lambda b, i: (b, 2 * i, 0)),
            pl.BlockSpec((1, _SB, _H), lambda b, i: (b, 2 * i + 1, 0)),
            pl.BlockSpec((2, _H), lambda b, i: (0, 0)),
            pl.BlockSpec((1, 1, 2 * _SB), lambda b, i: (b, 0, i)),
        ],
        out_specs=[
            pl.BlockSpec((1, 2, 2 * _SB), lambda b, i: (b, 0, i)),
            pl.BlockSpec((1, 2, 128), lambda b, i: (b, 0, 0)),
        ],
        out_shape=[
            jax.ShapeDtypeStruct((_B, 2, _S), jnp.float32),
            jax.ShapeDtypeStruct((_B, 2, 128), jnp.float32),
        ],
        scratch_shapes=[
            pltpu.VMEM((8, 128), jnp.float32),
            pltpu.VMEM((8, 128), jnp.float32),
        ],
        compiler_params=pltpu.CompilerParams(
            dimension_semantics=("parallel", "arbitrary")),
    )(seq_output, seq_output, w2, mask3)


def _topk_sc_body(logits_hbm, outv_hbm, outi_hbm, row_v, vout_v, iout_v):
    cid = lax.axis_index("c")
    sid = lax.axis_index("s")
    wid = sid * 2 + cid

    @pl.when(wid < _NJ)
    def _():
        pltpu.sync_copy(logits_hbm.at[wid], row_v)
        lane = lax.iota(jnp.int32, 16)
        U = 8

        def body(j, carry):
            ts = list(carry[0])
            js = list(carry[1])
            for u in range(U):
                base = (j * U + u) * 16
                v = row_v[pl.ds(base, 16)]
                gi = base + lane
                for r in range(_K):
                    upd = v > ts[r]
                    tv, ti = ts[r], js[r]
                    ts[r] = jnp.where(upd, v, tv)
                    js[r] = jnp.where(upd, gi, ti)
                    v = jnp.where(upd, tv, v)
                    gi = jnp.where(upd, ti, gi)
            return tuple(ts), tuple(js)

        init = (tuple(jnp.full((16,), _NEG, jnp.float32) for _ in range(_K)),
                tuple(jnp.zeros((16,), jnp.int32) for _ in range(_K)))
        ts, js = lax.fori_loop(0, _S // 16 // U, body, init)
        ts, js = list(ts), list(js)

        topv = jnp.full((16,), _NEG, jnp.float32)
        topi = jnp.zeros((16,), jnp.int32)
        for r in range(_K):
            m = ts[0]
            for q in range(1, _K):
                m = jnp.maximum(m, ts[q])
            smax = jnp.max(m)
            cand = jnp.full((16,), _IMAX, jnp.int32)
            for q in range(_K):
                cand = jnp.minimum(cand, jnp.where(ts[q] == smax, js[q], _IMAX))
            widx = jnp.min(cand)
            for q in range(_K):
                ts[q] = jnp.where(js[q] == widx, _NEG, ts[q])
            topv = jnp.where(lane == r, smax, topv)
            topi = jnp.where(lane == r, widx, topi)
        vout_v[...] = topv
        iout_v[...] = topi
        pltpu.sync_copy(vout_v, outv_hbm.at[wid])
        pltpu.sync_copy(iout_v, outi_hbm.at[wid])


@functools.cache
def _topk_sc():
    return functools.partial(
        pl.kernel,
        mesh=plsc.VectorSubcoreMesh(core_axis_name="c", subcore_axis_name="s"),
        out_type=[
            jax.ShapeDtypeStruct((_NJ, 16), jnp.float32),
            jax.ShapeDtypeStruct((_NJ, 16), jnp.int32),
        ],
        scratch_types=[
            pltpu.VMEM((_S,), jnp.float32),
            pltpu.VMEM((16,), jnp.float32),
            pltpu.VMEM((16,), jnp.int32),
        ],
        compiler_params=pltpu.CompilerParams(needs_layout_passes=False),
    )(_topk_sc_body)


def kernel(seq_output, cls_output, passage_mask, start_position,
           start_w, start_b, end_w, end_b):
    w2 = jnp.concatenate([start_w, end_w], axis=1).T
    mask3 = passage_mask[:, None, :]
    logits, lse = _project(seq_output, w2, mask3)
    lse = lse[:, :, 0]
    topv, topi = _topk_sc()(logits.reshape(_NJ, _S))
    topv = topv[:, :_K].reshape(_B, 2, _K)
    topi = topi[:, :_K].reshape(_B, 2, _K)
    logp = topv - lse[:, :, None]
    res = jnp.stack([logp, topi.astype(jnp.float32)], axis=-1)
    return res[:, 0], res[:, 1]


# speedup vs baseline: 1.0846x; 1.0177x over previous
import functools

import jax
import jax.numpy as jnp
import numpy as np
from jax import lax
from jax.experimental import pallas as pl
from jax.experimental.pallas import tpu as pltpu
from jax.experimental.pallas import tpu_sc as plsc

_B, _S, _H = 4, 8192, 1024
_SB = 1024
_NB2 = _S // (2 * _SB)
_K = 5
_NJ = _B * 2
_NEG = np.float32(-3.0e38)
_IMAX = np.int32(2**31 - 1)


def _proj_body(x1_ref, x2_ref, w_ref, mask_ref, out_ref, lse_ref, m_scr, s_scr):
    i = pl.program_id(1)
    w = w_ref[...]
    la = lax.dot_general(w, x1_ref[0], (((1,), (1,)), ((), ())),
                         preferred_element_type=jnp.float32)
    lb = lax.dot_general(w, x2_ref[0], (((1,), (1,)), ((), ())),
                         preferred_element_type=jnp.float32)
    logits = jnp.concatenate([la, lb], axis=1)
    mask = mask_ref[0]
    logits = logits + (1.0 - mask) * np.float32(-1e30)
    out_ref[0] = logits

    bm = jnp.max(logits, axis=1, keepdims=True)
    bs = jnp.sum(jnp.exp(logits - bm), axis=1, keepdims=True)
    bm = jnp.broadcast_to(bm, (2, 128))
    bs = jnp.broadcast_to(bs, (2, 128))

    @pl.when(i == 0)
    def _():
        m_scr[0:2, :] = bm
        s_scr[0:2, :] = bs

    @pl.when(i > 0)
    def _():
        m_old = m_scr[0:2, :]
        s_old = s_scr[0:2, :]
        m_new = jnp.maximum(m_old, bm)
        s_new = s_old * jnp.exp(m_old - m_new) + bs * jnp.exp(bm - m_new)
        m_scr[0:2, :] = m_new
        s_scr[0:2, :] = s_new

    @pl.when(i == _NB2 - 1)
    def _():
        lse_ref[0] = m_scr[0:2, :] + jnp.log(s_scr[0:2, :])


def _project(seq_output, w2, mask3):
    return pl.pallas_call(
        _proj_body,
        grid=(_B, _NB2),
        in_specs=[
            pl.BlockSpec((1, _SB, _H), lambda b, i: (b, 2 * i, 0)),
            pl.BlockSpec((1, _SB, _H), lambda b, i: (b, 2 * i + 1, 0)),
            pl.BlockSpec((2, _H), lambda b, i: (0, 0)),
            pl.BlockSpec((1, 1, 2 * _SB), lambda b, i: (b, 0, i)),
        ],
        out_specs=[
            pl.BlockSpec((1, 2, 2 * _SB), lambda b, i: (b, 0, i)),
            pl.BlockSpec((1, 2, 128), lambda b, i: (b, 0, 0)),
        ],
        out_shape=[
            jax.ShapeDtypeStruct((_B, 2, _S), jnp.float32),
            jax.ShapeDtypeStruct((_B, 2, 128), jnp.float32),
        ],
        scratch_shapes=[
            pltpu.VMEM((8, 128), jnp.float32),
            pltpu.VMEM((8, 128), jnp.float32),
        ],
        compiler_params=pltpu.CompilerParams(
            dimension_semantics=("parallel", "arbitrary")),
    )(seq_output, seq_output, w2, mask3)


_QS = _S // 4


def _topk_sc_body(logits_hbm, outv_hbm, outi_hbm, row_v, vout_v, iout_v,
                  m4v, m4i, sharedv, sharedi):
    cid = lax.axis_index("c")
    sid = lax.axis_index("s")
    row = cid * 4 + sid // 4
    part = sid % 4
    jid = row * 4 + part

    pltpu.sync_copy(logits_hbm.at[jid], row_v)
    lane = lax.iota(jnp.int32, 16)
    base0 = part * _QS
    U = 8

    def body(j, carry):
        ts = list(carry[0])
        js = list(carry[1])
        for u in range(U):
            off = (j * U + u) * 16
            v = row_v[pl.ds(off, 16)]
            gi = base0 + off + lane
            for r in range(_K):
                upd = v > ts[r]
                tv, ti = ts[r], js[r]
                ts[r] = jnp.where(upd, v, tv)
                js[r] = jnp.where(upd, gi, ti)
                v = jnp.where(upd, tv, v)
                gi = jnp.where(upd, ti, gi)
        return tuple(ts), tuple(js)

    init = (tuple(jnp.full((16,), _NEG, jnp.float32) for _ in range(_K)),
            tuple(jnp.zeros((16,), jnp.int32) for _ in range(_K)))
    ts, js = lax.fori_loop(0, _QS // 16 // U, body, init)
    ts, js = list(ts), list(js)

    topv = jnp.full((16,), _NEG, jnp.float32)
    topi = jnp.zeros((16,), jnp.int32)
    for r in range(_K):
        m = ts[0]
        for q in range(1, _K):
            m = jnp.maximum(m, ts[q])
        smax = jnp.max(m)
        cand = jnp.full((16,), _IMAX, jnp.int32)
        for q in range(_K):
            cand = jnp.minimum(cand, jnp.where(ts[q] == smax, js[q], _IMAX))
        widx = jnp.min(cand)
        for q in range(_K):
            ts[q] = jnp.where(js[q] == widx, _NEG, ts[q])
        topv = jnp.where(lane == r, smax, topv)
        topi = jnp.where(lane == r, widx, topi)
    vout_v[...] = topv
    iout_v[...] = topi

    pltpu.sync_copy(vout_v, sharedv.at[pl.ds(sid * 16, 16)])
    pltpu.sync_copy(iout_v, sharedi.at[pl.ds(sid * 16, 16)])
    plsc.subcore_barrier()

    @pl.when(part == 0)
    def _():
        pltpu.sync_copy(sharedv.at[pl.ds(sid * 16, 64)], m4v)
        pltpu.sync_copy(sharedi.at[pl.ds(sid * 16, 64)], m4i)
        vs = [m4v[pl.ds(k * 16, 16)] for k in range(4)]
        is_ = [m4i[pl.ds(k * 16, 16)] for k in range(4)]
        gtv = jnp.full((16,), _NEG, jnp.float32)
        gti = jnp.zeros((16,), jnp.int32)
        for r in range(_K):
            m = vs[0]
            for k in range(1, 4):
                m = jnp.maximum(m, vs[k])
            smax = jnp.max(m)
            cand = jnp.full((16,), _IMAX, jnp.int32)
            for k in range(4):
                cand = jnp.minimum(cand, jnp.where(vs[k] == smax, is_[k], _IMAX))
            widx = jnp.min(cand)
            for k in range(4):
                vs[k] = jnp.where(is_[k] == widx, _NEG, vs[k])
            gtv = jnp.where(lane == r, smax, gtv)
            gti = jnp.where(lane == r, widx, gti)
        vout_v[...] = gtv
        iout_v[...] = gti
        pltpu.sync_copy(vout_v, outv_hbm.at[row])
        pltpu.sync_copy(iout_v, outi_hbm.at[row])


@functools.cache
def _topk_sc():
    return functools.partial(
        pl.kernel,
        mesh=plsc.VectorSubcoreMesh(core_axis_name="c", subcore_axis_name="s"),
        out_type=[
            jax.ShapeDtypeStruct((_NJ, 16), jnp.float32),
            jax.ShapeDtypeStruct((_NJ, 16), jnp.int32),
        ],
        scratch_types=[
            pltpu.VMEM((_QS,), jnp.float32),
            pltpu.VMEM((16,), jnp.float32),
            pltpu.VMEM((16,), jnp.int32),
            pltpu.VMEM((64,), jnp.float32),
            pltpu.VMEM((64,), jnp.int32),
            pltpu.VMEM_SHARED((256,), jnp.float32),
            pltpu.VMEM_SHARED((256,), jnp.int32),
        ],
        compiler_params=pltpu.CompilerParams(needs_layout_passes=False),
    )(_topk_sc_body)


def kernel(seq_output, cls_output, passage_mask, start_position,
           start_w, start_b, end_w, end_b):
    w2 = jnp.concatenate([start_w, end_w], axis=1).T
    mask3 = passage_mask[:, None, :]
    logits, lse = _project(seq_output, w2, mask3)
    lse = lse[:, :, 0]
    topv, topi = _topk_sc()(logits.reshape(_NJ * 4, _QS))
    topv = topv[:, :_K].reshape(_B, 2, _K)
    topi = topi[:, :_K].reshape(_B, 2, _K)
    logp = topv - lse[:, :, None]
    res = jnp.stack([logp, topi.astype(jnp.float32)], axis=-1)
    return res[:, 0], res[:, 1]
